# baseline (speedup 1.0000x reference)
import functools

import jax
import jax.numpy as jnp
from jax.experimental import pallas as pl
from jax.experimental.pallas import tpu as pltpu

_SH = 16
_SW = 16
_BIG = 1e16
_SLAB = 56
_SROWS = 52
_SPAD = 200


def _rel_offsets(nsw):
    return [d + r for d in (-nsw, 0, nsw) for r in (-1, 0, 1)]


def _slab_start(bi, nsw):
    return 8 * jnp.clip((bi * nsw - nsw - 1) // 8, 0, (_SPAD - _SLAB) // 8)


def _neg_bias(bi, start, nsw, nspix, W):
    so = start + jax.lax.broadcasted_iota(jnp.int32, (_SLAB, W), 0)
    base = bi * nsw + jax.lax.broadcasted_iota(jnp.int32, (_SLAB, W), 1) // _SW
    member = jnp.zeros((_SLAB, W), jnp.bool_)
    for r in _rel_offsets(nsw):
        cand = base + r
        member = member | ((so == cand) & (cand < nspix))
    return jnp.where(member, 0.0, -_BIG)


def _means_body(scale_ref, x_ref, o_ref, pn_ref, *, C, nsw, W, FW):
    s = scale_ref[0, 0]
    xf = x_ref[0, :, 0, 0, :]
    pn_ref[0, 0] = jnp.sum(xf * xf, axis=0, keepdims=True)
    lio = jax.lax.broadcasted_iota(jnp.int32, (FW, nsw), 0)
    bjo = jax.lax.broadcasted_iota(jnp.int32, (FW, nsw), 1)
    selT = jnp.where((lio % W) // _SW == bjo, 1.0, 0.0)
    m = jax.lax.dot_general(
        xf, selT, (((1,), (0,)), ((), ())), preferred_element_type=jnp.float32,
        precision=jax.lax.Precision.HIGHEST)
    o_ref[0, 0] = m.T * (s / float(_SH * _SW))


def _flat_aff(sp, sn, xf, pn, bi, start, s, *, nsw, nspix, W, FW):
    dots = jax.lax.dot_general(
        sp, xf, (((1,), (0,)), ((), ())), preferred_element_type=jnp.float32)
    snb224 = _neg_bias(bi, start, nsw, nspix, W) - sn
    snb = jnp.concatenate([snb224] * (FW // W), axis=1)
    neg = (2.0 * s) * dots + (snb - (s * s) * pn)
    mx = jnp.max(neg, axis=0, keepdims=True)
    e = jnp.exp(neg - mx)
    return e / jnp.sum(e, axis=0, keepdims=True)


def _iter0_body(scale_ref, x_ref, sp_ref, pn_ref, o_ref, acc_ref, den_ref, *,
                C, nsw, nsh, nspix, W, FW):
    bi = pl.program_id(1)

    @pl.when(bi == 0)
    def _init():
        acc_ref[...] = jnp.zeros_like(acc_ref)
        den_ref[...] = jnp.zeros_like(den_ref)

    s = scale_ref[0, 0]
    start = _slab_start(bi, nsw)
    sp = sp_ref[0, pl.ds(start, _SLAB), :]
    sn = jnp.sum(sp * sp, axis=1, keepdims=True)
    xf = x_ref[0, :, 0, 0, :]
    aff = _flat_aff(sp, sn, xf, pn_ref[0, 0], bi, start, s,
                    nsw=nsw, nspix=nspix, W=W, FW=FW)
    acc_ref[pl.ds(start, _SLAB), :] += jax.lax.dot_general(
        aff, xf, (((1,), (1,)), ((), ())), preferred_element_type=jnp.float32)
    den_ref[pl.ds(start, _SLAB), :] += jnp.sum(aff, axis=1, keepdims=True)

    @pl.when(bi == nsh - 1)
    def _finish():
        o_ref[0] = (acc_ref[...] * s) / (den_ref[...] + 1e-16)


def _iter1_body(scale_ref, x_ref, sp_ref, pn_ref, o_ref, *, C, nsw, nspix, W, FW):
    bi = pl.program_id(1)
    s = scale_ref[0, 0]
    start = _slab_start(bi, nsw)
    sp = sp_ref[0, pl.ds(start, _SLAB), :]
    sn = jnp.sum(sp * sp, axis=1, keepdims=True)
    xf = x_ref[0, :, 0, 0, :]
    aff = _flat_aff(sp, sn, xf, pn_ref[0, 0], bi, start, s,
                    nsw=nsw, nspix=nspix, W=W, FW=FW)
    o_ref[0] = jnp.zeros((nspix, 1, _SH, W), jnp.float32)
    r3 = aff.reshape(_SLAB, _SH, W)
    o_ref[0, pl.ds(start, _SROWS), 0, :, :] = r3[:_SROWS]


def kernel(x, stoken_size):
    B, C, H, W = x.shape
    nsh, nsw = H // _SH, W // _SW
    nspix = nsh * nsw
    FW = _SH * W
    scale = (jnp.asarray(stoken_size, jnp.float32) / float(_SH)).reshape(1, 1)
    xflat = x.reshape(B, C, nsh, 1, FW)

    x_spec = pl.BlockSpec((1, C, 1, 1, FW), lambda b, bi: (b, 0, bi, 0, 0))
    scale_spec = pl.BlockSpec((1, 1), lambda b, bi: (0, 0))
    sp_spec = pl.BlockSpec((1, _SPAD, C), lambda b, bi: (b, 0, 0))
    pn_spec = pl.BlockSpec((1, 1, 1, FW), lambda b, bi: (b, bi, 0, 0))
    seq = pltpu.CompilerParams(dimension_semantics=("arbitrary", "arbitrary"))

    sp0, pn = pl.pallas_call(
        functools.partial(_means_body, C=C, nsw=nsw, W=W, FW=FW),
        grid=(B, nsh),
        in_specs=[scale_spec, x_spec],
        out_specs=[
            pl.BlockSpec((1, 1, nsw, C), lambda b, bi: (b, bi, 0, 0)),
            pn_spec,
        ],
        out_shape=[
            jax.ShapeDtypeStruct((B, nsh, nsw, C), jnp.float32),
            jax.ShapeDtypeStruct((B, nsh, 1, FW), jnp.float32),
        ],
        compiler_params=seq,
    )(scale, xflat)

    sp0p = jnp.concatenate(
        [sp0.reshape(B, nspix, C),
         jnp.zeros((B, _SPAD - nspix, C), jnp.float32)], axis=1)

    sp1 = pl.pallas_call(
        functools.partial(_iter0_body, C=C, nsw=nsw, nsh=nsh, nspix=nspix, W=W, FW=FW),
        grid=(B, nsh),
        in_specs=[scale_spec, x_spec, sp_spec, pn_spec],
        out_specs=pl.BlockSpec((1, _SPAD, C), lambda b, bi: (b, 0, 0)),
        out_shape=jax.ShapeDtypeStruct((B, _SPAD, C), jnp.float32),
        scratch_shapes=[
            pltpu.VMEM((_SPAD, C), jnp.float32),
            pltpu.VMEM((_SPAD, 1), jnp.float32),
        ],
        compiler_params=seq,
    )(scale, xflat, sp0p, pn)

    aff = pl.pallas_call(
        functools.partial(_iter1_body, C=C, nsw=nsw, nspix=nspix, W=W, FW=FW),
        grid=(B, nsh),
        in_specs=[scale_spec, x_spec, sp_spec, pn_spec],
        out_specs=pl.BlockSpec((1, nspix, 1, _SH, W), lambda b, bi: (b, 0, bi, 0, 0)),
        out_shape=jax.ShapeDtypeStruct((B, nspix, nsh, _SH, W), jnp.float32),
        compiler_params=seq,
    )(scale, xflat, sp1, pn)

    return aff.reshape(B, nspix, H * W), nspix


# speedup vs baseline: 1.3335x; 1.3335x over previous
import functools

import jax
import jax.numpy as jnp
from jax.experimental import pallas as pl
from jax.experimental.pallas import tpu as pltpu

_SH = 16
_SW = 16
_BIG = 1e16
_SLAB = 56
_SROWS = 52
_SPAD = 200


def _rel_offsets(nsw):
    return [d + r for d in (-nsw, 0, nsw) for r in (-1, 0, 1)]


def _slab_start(bi, nsw):
    return 8 * jnp.clip((bi * nsw - nsw - 1) // 8, 0, (_SPAD - _SLAB) // 8)


def _neg_bias(bi, start, nsw, nspix, W):
    so = start + jax.lax.broadcasted_iota(jnp.int32, (_SLAB, W), 0)
    base = bi * nsw + jax.lax.broadcasted_iota(jnp.int32, (_SLAB, W), 1) // _SW
    member = jnp.zeros((_SLAB, W), jnp.bool_)
    for r in _rel_offsets(nsw):
        cand = base + r
        member = member | ((so == cand) & (cand < nspix))
    return jnp.where(member, 0.0, -_BIG)


def _row_aff(sp, xr, snb, pn, s1, s2):
    dots = jax.lax.dot_general(
        sp, xr, (((1,), (0,)), ((), ())), preferred_element_type=jnp.float32)
    neg = s1 * dots + (snb - s2 * pn)
    mx = jnp.max(neg, axis=0, keepdims=True)
    e = jnp.exp(neg - mx)
    return e / jnp.sum(e, axis=0, keepdims=True)


def _means_body(scale_ref, x_ref, o_ref, pn_ref, *, C, nsw, W):
    s = scale_ref[0, 0]
    xb = x_ref[0]
    pn_ref[0, 0] = jnp.sum(xb * xb, axis=0)
    xs = xb[:, 0, :]
    for h in range(1, _SH):
        xs = xs + xb[:, h, :]
    wio = jax.lax.broadcasted_iota(jnp.int32, (W, nsw), 0) // _SW
    bjo = jax.lax.broadcasted_iota(jnp.int32, (W, nsw), 1)
    selT = jnp.where(wio == bjo, 1.0, 0.0)
    m = jax.lax.dot_general(
        xs, selT, (((1,), (0,)), ((), ())), preferred_element_type=jnp.float32,
        precision=jax.lax.Precision.HIGHEST)
    o_ref[0, 0] = m.T * (s / float(_SH * _SW))


def _iter0_body(scale_ref, x_ref, sp_ref, pn_ref, o_ref, acc_ref, den_ref, *,
                C, nsw, nsh, nspix, W):
    bi = pl.program_id(1)

    @pl.when(bi == 0)
    def _init():
        acc_ref[...] = jnp.zeros_like(acc_ref)
        den_ref[...] = jnp.zeros_like(den_ref)

    s = scale_ref[0, 0]
    start = _slab_start(bi, nsw)
    sp = sp_ref[0, pl.ds(start, _SLAB), :]
    sn = jnp.sum(sp * sp, axis=1, keepdims=True)
    snb = _neg_bias(bi, start, nsw, nspix, W) - sn
    xb = x_ref[0]
    pn_all = pn_ref[0, 0]
    acc = jnp.zeros((_SLAB, C), jnp.float32)
    den = jnp.zeros((_SLAB, 1), jnp.float32)
    for h in range(_SH):
        xr = xb[:, h, :]
        aff = _row_aff(sp, xr, snb, pn_all[h:h + 1, :], 2.0 * s, s * s)
        acc = acc + jax.lax.dot_general(
            aff, xr, (((1,), (1,)), ((), ())), preferred_element_type=jnp.float32)
        den = den + jnp.sum(aff, axis=1, keepdims=True)
    acc_ref[pl.ds(start, _SLAB), :] += acc
    den_ref[pl.ds(start, _SLAB), :] += den

    @pl.when(bi == nsh - 1)
    def _finish():
        o_ref[0] = (acc_ref[...] * s) / (den_ref[...] + 1e-16)


def _iter1_body(scale_ref, x_ref, sp_ref, pn_ref, o_ref, *, C, nsw, nspix, W):
    bi = pl.program_id(1)
    s = scale_ref[0, 0]
    start = _slab_start(bi, nsw)
    sp = sp_ref[0, pl.ds(start, _SLAB), :]
    sn = jnp.sum(sp * sp, axis=1, keepdims=True)
    snb = _neg_bias(bi, start, nsw, nspix, W) - sn
    xb = x_ref[0]
    pn_all = pn_ref[0, 0]
    o_ref[0] = jnp.zeros((nspix, _SH, W), jnp.float32)
    for h in range(_SH):
        aff = _row_aff(sp, xb[:, h, :], snb, pn_all[h:h + 1, :], 2.0 * s, s * s)
        o_ref[0, pl.ds(start, _SROWS), h, :] = aff[:_SROWS]


def kernel(x, stoken_size):
    B, C, H, W = x.shape
    nsh, nsw = H // _SH, W // _SW
    nspix = nsh * nsw
    scale = (jnp.asarray(stoken_size, jnp.float32) / float(_SH)).reshape(1, 1)

    x_spec = pl.BlockSpec((1, C, _SH, W), lambda b, bi: (b, 0, bi, 0))
    scale_spec = pl.BlockSpec((1, 1), lambda b, bi: (0, 0))
    sp_spec = pl.BlockSpec((1, _SPAD, C), lambda b, bi: (b, 0, 0))
    pn_spec = pl.BlockSpec((1, 1, _SH, W), lambda b, bi: (b, bi, 0, 0))
    seq = pltpu.CompilerParams(dimension_semantics=("arbitrary", "arbitrary"))

    sp0, pn = pl.pallas_call(
        functools.partial(_means_body, C=C, nsw=nsw, W=W),
        grid=(B, nsh),
        in_specs=[scale_spec, x_spec],
        out_specs=[
            pl.BlockSpec((1, 1, nsw, C), lambda b, bi: (b, bi, 0, 0)),
            pn_spec,
        ],
        out_shape=[
            jax.ShapeDtypeStruct((B, nsh, nsw, C), jnp.float32),
            jax.ShapeDtypeStruct((B, nsh, _SH, W), jnp.float32),
        ],
        compiler_params=seq,
    )(scale, x)

    sp0p = jnp.concatenate(
        [sp0.reshape(B, nspix, C),
         jnp.zeros((B, _SPAD - nspix, C), jnp.float32)], axis=1)

    sp1 = pl.pallas_call(
        functools.partial(_iter0_body, C=C, nsw=nsw, nsh=nsh, nspix=nspix, W=W),
        grid=(B, nsh),
        in_specs=[scale_spec, x_spec, sp_spec, pn_spec],
        out_specs=pl.BlockSpec((1, _SPAD, C), lambda b, bi: (b, 0, 0)),
        out_shape=jax.ShapeDtypeStruct((B, _SPAD, C), jnp.float32),
        scratch_shapes=[
            pltpu.VMEM((_SPAD, C), jnp.float32),
            pltpu.VMEM((_SPAD, 1), jnp.float32),
        ],
        compiler_params=seq,
    )(scale, x, sp0p, pn)

    aff = pl.pallas_call(
        functools.partial(_iter1_body, C=C, nsw=nsw, nspix=nspix, W=W),
        grid=(B, nsh),
        in_specs=[scale_spec, x_spec, sp_spec, pn_spec],
        out_specs=pl.BlockSpec((1, nspix, _SH, W), lambda b, bi: (b, 0, bi, 0)),
        out_shape=jax.ShapeDtypeStruct((B, nspix, H, W), jnp.float32),
        compiler_params=seq,
    )(scale, x, sp1, pn)

    return aff.reshape(B, nspix, H * W), nspix


# speedup vs baseline: 4.1411x; 3.1054x over previous
import functools

import jax
import jax.numpy as jnp
from jax.experimental import pallas as pl
from jax.experimental.pallas import tpu as pltpu

_SH = 16
_SW = 16
_BIG = 1e16
_SLAB = 56
_SROWS = 52
_SPAD = 200


def _rel_offsets(nsw):
    return [d + r for d in (-nsw, 0, nsw) for r in (-1, 0, 1)]


def _slab_start(bi, nsw):
    return 8 * jnp.clip((bi * nsw - nsw - 1) // 8, 0, (_SPAD - _SLAB) // 8)


def _neg_bias(bi, start, nsw, nspix, W):
    so = start + jax.lax.broadcasted_iota(jnp.int32, (_SLAB, W), 0)
    base = bi * nsw + jax.lax.broadcasted_iota(jnp.int32, (_SLAB, W), 1) // _SW
    member = jnp.zeros((_SLAB, W), jnp.bool_)
    for r in _rel_offsets(nsw):
        cand = base + r
        member = member | ((so == cand) & (cand < nspix))
    return jnp.where(member, 0.0, -_BIG)


def _row_aff(sp, xr, snb, pn, s1, s2):
    dots = jax.lax.dot_general(
        sp, xr, (((1,), (0,)), ((), ())), preferred_element_type=jnp.float32)
    neg = s1 * dots + (snb - s2 * pn)
    mx = jnp.max(neg, axis=0, keepdims=True)
    e = jnp.exp(neg - mx)
    return e / jnp.sum(e, axis=0, keepdims=True)


def _means_body(scale_ref, x_ref, o_ref, pn_ref, *, C, nsw, W):
    s = scale_ref[0, 0]
    xb = x_ref[0]
    pn_ref[0, 0] = jnp.sum(xb * xb, axis=0)
    xs = xb[:, 0, :]
    for h in range(1, _SH):
        xs = xs + xb[:, h, :]
    wio = jax.lax.broadcasted_iota(jnp.int32, (W, nsw), 0) // _SW
    bjo = jax.lax.broadcasted_iota(jnp.int32, (W, nsw), 1)
    selT = jnp.where(wio == bjo, 1.0, 0.0)
    m = jax.lax.dot_general(
        xs, selT, (((1,), (0,)), ((), ())), preferred_element_type=jnp.float32,
        precision=jax.lax.Precision.HIGHEST)
    o_ref[0, 0] = m.T * (s / float(_SH * _SW))


def _iter0_body(scale_ref, x_ref, sp_ref, pn_ref, o_ref, acc_ref, den_ref, *,
                C, nsw, nsh, nspix, W):
    bi = pl.program_id(1)

    @pl.when(bi == 0)
    def _init():
        acc_ref[...] = jnp.zeros_like(acc_ref)
        den_ref[...] = jnp.zeros_like(den_ref)

    s = scale_ref[0, 0]
    start = _slab_start(bi, nsw)
    sp = sp_ref[0, pl.ds(start, _SLAB), :]
    sn = jnp.sum(sp * sp, axis=1, keepdims=True)
    snb = _neg_bias(bi, start, nsw, nspix, W) - sn
    xb = x_ref[0]
    pn_all = pn_ref[0, 0]
    acc = jnp.zeros((_SLAB, C), jnp.float32)
    den = jnp.zeros((_SLAB, 1), jnp.float32)
    for h in range(_SH):
        xr = xb[:, h, :]
        aff = _row_aff(sp, xr, snb, pn_all[h:h + 1, :], 2.0 * s, s * s)
        acc = acc + jax.lax.dot_general(
            aff, xr, (((1,), (1,)), ((), ())), preferred_element_type=jnp.float32)
        den = den + jnp.sum(aff, axis=1, keepdims=True)
    acc_ref[pl.ds(start, _SLAB), :] += acc
    den_ref[pl.ds(start, _SLAB), :] += den

    @pl.when(bi == nsh - 1)
    def _finish():
        o_ref[0] = (acc_ref[...] * s) / (den_ref[...] + 1e-16)


def _iter1_body(scale_ref, x_ref, sp_ref, pn_ref, o_ref, *, C, nsw, nspix, W):
    bi = pl.program_id(1)
    s = scale_ref[0, 0]
    start = _slab_start(bi, nsw)
    sp = sp_ref[0, pl.ds(start, _SLAB), :]
    sn = jnp.sum(sp * sp, axis=1, keepdims=True)
    snb = _neg_bias(bi, start, nsw, nspix, W) - sn
    xb = x_ref[0]
    pn_all = pn_ref[0, 0]
    o_ref[0] = jnp.zeros((nspix, _SH * W), jnp.float32)
    for h in range(_SH):
        aff = _row_aff(sp, xb[:, h, :], snb, pn_all[h:h + 1, :], 2.0 * s, s * s)
        o_ref[0, pl.ds(start, _SROWS), h * W:(h + 1) * W] = aff[:_SROWS]


def kernel(x, stoken_size):
    B, C, H, W = x.shape
    nsh, nsw = H // _SH, W // _SW
    nspix = nsh * nsw
    scale = (jnp.asarray(stoken_size, jnp.float32) / float(_SH)).reshape(1, 1)

    x_spec = pl.BlockSpec((1, C, _SH, W), lambda b, bi: (b, 0, bi, 0))
    scale_spec = pl.BlockSpec((1, 1), lambda b, bi: (0, 0))
    sp_spec = pl.BlockSpec((1, _SPAD, C), lambda b, bi: (b, 0, 0))
    pn_spec = pl.BlockSpec((1, 1, _SH, W), lambda b, bi: (b, bi, 0, 0))
    seq = pltpu.CompilerParams(dimension_semantics=("arbitrary", "arbitrary"))

    sp0, pn = pl.pallas_call(
        functools.partial(_means_body, C=C, nsw=nsw, W=W),
        grid=(B, nsh),
        in_specs=[scale_spec, x_spec],
        out_specs=[
            pl.BlockSpec((1, 1, nsw, C), lambda b, bi: (b, bi, 0, 0)),
            pn_spec,
        ],
        out_shape=[
            jax.ShapeDtypeStruct((B, nsh, nsw, C), jnp.float32),
            jax.ShapeDtypeStruct((B, nsh, _SH, W), jnp.float32),
        ],
        compiler_params=seq,
    )(scale, x)

    sp0p = jnp.concatenate(
        [sp0.reshape(B, nspix, C),
         jnp.zeros((B, _SPAD - nspix, C), jnp.float32)], axis=1)

    sp1 = pl.pallas_call(
        functools.partial(_iter0_body, C=C, nsw=nsw, nsh=nsh, nspix=nspix, W=W),
        grid=(B, nsh),
        in_specs=[scale_spec, x_spec, sp_spec, pn_spec],
        out_specs=pl.BlockSpec((1, _SPAD, C), lambda b, bi: (b, 0, 0)),
        out_shape=jax.ShapeDtypeStruct((B, _SPAD, C), jnp.float32),
        scratch_shapes=[
            pltpu.VMEM((_SPAD, C), jnp.float32),
            pltpu.VMEM((_SPAD, 1), jnp.float32),
        ],
        compiler_params=seq,
    )(scale, x, sp0p, pn)

    aff = pl.pallas_call(
        functools.partial(_iter1_body, C=C, nsw=nsw, nspix=nspix, W=W),
        grid=(B, nsh),
        in_specs=[scale_spec, x_spec, sp_spec, pn_spec],
        out_specs=pl.BlockSpec((1, nspix, _SH * W), lambda b, bi: (b, 0, bi)),
        out_shape=jax.ShapeDtypeStruct((B, nspix, H * W), jnp.float32),
        compiler_params=seq,
    )(scale, x, sp1, pn)

    return aff, nspix
